# baseline (speedup 1.0000x reference)
import jax
import jax.numpy as jnp
from jax import lax
from jax.experimental import pallas as pl
from jax.experimental.pallas import tpu as pltpu
from jax.experimental.pallas import tpu_sc as plsc

_N = 10000
_E = 320000
_D = 128

_NC = 2
_NS = 16
_NWK = _NC * _NS
_CH = 128
_NCHK = _E // _CH
_CREM = _NCHK % _NWK
_CBASE = _NCHK // _NWK
_CMAX = _CBASE + 1
_BLK = 8
_PCAP = 80 * _CH
_NP = 10240
_RPT = _NP // _NS
_BR = 2000
_DUMP = _N + 64
_CROWS = _NP // 16


def _sc_compact_body(src2d_hbm, dst2d_hbm, fr_hbm, iota_hbm, csrc_hbm,
                     cdst_hbm, nch_hbm, cnt_out_hbm, fbuf, sall, dall, osrc,
                     odst, cbuf, iotab, cvec, cnt2):
    cid = lax.axis_index("c")
    sid = lax.axis_index("s")
    wid = sid * _NC + cid
    c0 = wid * _CBASE + lax.min(wid, _CREM)
    cnt = _CBASE + jnp.where(wid < _CREM, 1, 0)
    r0c = sid * (_CROWS // _NS)

    pltpu.sync_copy(fr_hbm, fbuf)
    pltpu.sync_copy(iota_hbm, iotab)
    pltpu.sync_copy(src2d_hbm.at[pl.ds(c0, _CMAX)], sall)
    pltpu.sync_copy(dst2d_hbm.at[pl.ds(c0, _CMAX)], dall)

    zsv = jnp.zeros((16,), jnp.float32)

    def zero_cbuf(i, carry):
        cbuf[i] = zsv
        return carry

    lax.fori_loop(0, _CROWS, zero_cbuf, 0)
    pltpu.sync_copy(cbuf.at[pl.ds(0, _CROWS // _NS)],
                    cnt2.at[pl.ds(r0c, _CROWS // _NS)])

    lane = lax.iota(jnp.int32, 16)

    def prefill(i, carry):
        spread = (lane + i * 16) & 127
        osrc[pl.ds(i * 16, 16)] = spread
        odst[pl.ds(i * 16, 16)] = _N + spread
        return carry

    lax.fori_loop(0, _PCAP // 16, prefill, 0)
    plsc.subcore_barrier()

    onev = jnp.ones((16,), jnp.float32)

    def chunk(j, cur):
        for t in range(_CH // 16):
            sv = sall[j, pl.ds(t * 16, 16)]
            dv = dall[j, pl.ds(t * 16, 16)]
            plsc.addupdate_scatter(cbuf, [dv >> 4, dv & 15], onev)
            fv = plsc.load_gather(fbuf, [sv])
            m = fv == 0
            plsc.store_compressed(osrc.at[pl.ds(cur, 16)], sv, mask=m)
            plsc.store_compressed(odst.at[pl.ds(cur, 16)], dv, mask=m)
            cur = cur + jnp.max(plsc.all_reduce_population_count(m))
        return cur

    cur = lax.fori_loop(0, cnt, chunk, jnp.int32(0))
    nblk = (cur + _BLK * _CH - 1) // (_BLK * _CH)

    for k in range(_CROWS // _CH):
        pltpu.sync_copy(cbuf.at[pl.ds(k * _CH, _CH)], cnt2.at[iotab.at[k]],
                        add=True)

    pltpu.sync_copy(osrc, csrc_hbm.at[wid])
    pltpu.sync_copy(odst, cdst_hbm.at[wid])
    cvec[...] = jnp.broadcast_to(nblk, (16,))
    pltpu.sync_copy(cvec, nch_hbm.at[wid])

    plsc.subcore_barrier()
    pltpu.sync_copy(cnt2.at[pl.ds(r0c, _CROWS // _NS)],
                    cnt_out_hbm.at[cid, pl.ds(r0c, _CROWS // _NS)])


def _sc_scatter_body(x_hbm, csrc_hbm, cdst_hbm, nch_hbm, zeros_hbm,
                     acc_out_hbm, sblks, dblks, rows, cvec,
                     isems, gsems, ssems, acc):
    cid = lax.axis_index("c")
    sid = lax.axis_index("s")
    wid = sid * _NC + cid
    r0 = sid * _RPT
    pltpu.sync_copy(zeros_hbm.at[pl.ds(r0, _RPT)], acc.at[pl.ds(r0, _RPT)])
    pltpu.sync_copy(nch_hbm.at[wid], cvec)
    plsc.subcore_barrier()

    nblk = jnp.max(cvec[...])

    def fetch(b, p):
        o = pl.multiple_of(b * _BLK * _CH, 8)
        pltpu.async_copy(csrc_hbm.at[wid, pl.ds(o, _BLK * _CH)], sblks[p],
                         isems[2 * p])
        pltpu.async_copy(cdst_hbm.at[wid, pl.ds(o, _BLK * _CH)], dblks[p],
                         isems[2 * p + 1])

    def fwait(p):
        pltpu.make_async_copy(csrc_hbm.at[wid, pl.ds(0, _BLK * _CH)],
                              sblks[p], isems[2 * p]).wait()
        pltpu.make_async_copy(cdst_hbm.at[wid, pl.ds(0, _BLK * _CH)],
                              dblks[p], isems[2 * p + 1]).wait()

    def process(p):
        sblk, dblk = sblks[p], dblks[p]

        def gather(u, q):
            return pltpu.async_copy(x_hbm.at[sblk.at[pl.ds(u * _CH, _CH)]],
                                    rows[q], gsems[q])

        def scatter(u, q):
            return pltpu.async_copy(rows[q], acc.at[dblk.at[pl.ds(u * _CH, _CH)]],
                                    ssems[q], add=True)

        gd, sc = {}, {}
        gd[0] = gather(0, 0)
        gd[1] = gather(1, 1)
        for u in range(0, _BLK, 2):
            gd[u].wait()
            sc[u] = scatter(u, 0)
            gd[u + 1].wait()
            sc[u + 1] = scatter(u + 1, 1)
            if u + 3 < _BLK:
                sc[u].wait()
                gd[u + 2] = gather(u + 2, 0)
                sc[u + 1].wait()
                gd[u + 3] = gather(u + 3, 1)
        sc[_BLK - 2].wait()
        sc[_BLK - 1].wait()

    @pl.when(nblk > 0)
    def _():
        fetch(0, 0)

        def pair(i, carry):
            b0 = 2 * i
            b1 = b0 + 1
            fwait(0)
            fetch(lax.min(b1, nblk - 1), 1)
            process(0)
            fwait(1)
            fetch(lax.min(b0 + 2, nblk - 1), 0)

            @pl.when(b1 < nblk)
            def _():
                process(1)

            return carry

        lax.fori_loop(0, (nblk + 1) // 2, pair, 0)
        fwait(0)

    plsc.subcore_barrier()
    pltpu.sync_copy(acc.at[pl.ds(r0, _RPT)], acc_out_hbm.at[cid, pl.ds(r0, _RPT)])


def _combine_body(acc_ref, cnt_ref, x_ref, f_ref, agg_ref, wn_ref, b_ref,
                  wr_ref, ffull_ref, out_ref):
    summed = acc_ref[0] + acc_ref[1]
    count = cnt_ref[0] + cnt_ref[1]
    mean = summed / jnp.maximum(count, 1.0)
    f = f_ref[...]
    xz = x_ref[...] * (1.0 - f)
    agg = agg_ref[...]
    use_hybrid = jnp.max(ffull_ref[...]) > 0.0
    target = (jnp.sum(jnp.abs(agg), axis=1, keepdims=True) > 0.0) & use_hybrid
    neigh_in = jnp.where(target, agg, mean)
    root_in = jnp.where(target, 0.0, xz)
    out_ref[...] = (
        jnp.dot(neigh_in, wn_ref[...], preferred_element_type=jnp.float32)
        + b_ref[...]
        + jnp.dot(root_in, wr_ref[...], preferred_element_type=jnp.float32))


def kernel(x, edge_index, frontier_mask, aggregated_neighbors,
           W_neigh, b_neigh, W_root):
    f = frontier_mask.astype(jnp.float32).reshape(_N, 1)
    fr_i = frontier_mask.astype(jnp.int32)
    src2d = jnp.pad(edge_index[0].reshape(_NCHK, _CH), ((0, 1), (0, 0)))
    dst2d = jnp.pad(edge_index[1].reshape(_NCHK, _CH), ((0, 1), (0, 0)))
    zeros = jnp.zeros((_NP, _D), jnp.float32)
    b2 = b_neigh.reshape(1, _D)

    mesh = plsc.VectorSubcoreMesh(core_axis_name="c", subcore_axis_name="s")
    params = pltpu.CompilerParams(use_tc_tiling_on_sc=False,
                                  needs_layout_passes=False)
    sc_compact = pl.kernel(
        _sc_compact_body,
        mesh=mesh,
        compiler_params=params,
        out_type=[jax.ShapeDtypeStruct((_NWK, _PCAP), jnp.int32),
                  jax.ShapeDtypeStruct((_NWK, _PCAP), jnp.int32),
                  jax.ShapeDtypeStruct((_NWK, 16), jnp.int32),
                  jax.ShapeDtypeStruct((_NC, _CROWS, 16), jnp.float32)],
        scratch_types=[
            pltpu.VMEM((_N,), jnp.int32),
            pltpu.VMEM((_CMAX, _CH), jnp.int32),
            pltpu.VMEM((_CMAX, _CH), jnp.int32),
            pltpu.VMEM((_PCAP,), jnp.int32),
            pltpu.VMEM((_PCAP,), jnp.int32),
            pltpu.VMEM((_CROWS, 16), jnp.float32),
            pltpu.VMEM((_CROWS // _CH, _CH), jnp.int32),
            pltpu.VMEM((16,), jnp.int32),
            pltpu.VMEM_SHARED((_CROWS, 16), jnp.float32),
        ],
    )
    iota_rows = jnp.arange(_CROWS, dtype=jnp.int32).reshape(_CROWS // _CH, _CH)
    csrc, cdst, nch, cnt2 = sc_compact(src2d, dst2d, fr_i, iota_rows)

    sc_scatter = pl.kernel(
        _sc_scatter_body,
        mesh=mesh,
        compiler_params=params,
        out_type=jax.ShapeDtypeStruct((_NC, _NP, _D), jnp.float32),
        scratch_types=[
            [pltpu.VMEM((_BLK * _CH,), jnp.int32) for _ in range(2)],
            [pltpu.VMEM((_BLK * _CH,), jnp.int32) for _ in range(2)],
            [pltpu.VMEM((_CH, _D), jnp.float32) for _ in range(2)],
            pltpu.VMEM((16,), jnp.int32),
            [pltpu.SemaphoreType.DMA for _ in range(4)],
            [pltpu.SemaphoreType.DMA for _ in range(2)],
            [pltpu.SemaphoreType.DMA for _ in range(2)],
            pltpu.VMEM_SHARED((_NP, _D), jnp.float32),
        ],
    )
    acc = sc_scatter(x, csrc, cdst, nch, zeros)

    cnt3 = cnt2.reshape(_NC, _NP, 1)
    out = pl.pallas_call(
        _combine_body,
        grid=(_N // _BR,),
        in_specs=[
            pl.BlockSpec((_NC, _BR, _D), lambda i: (0, i, 0)),
            pl.BlockSpec((_NC, _BR, 1), lambda i: (0, i, 0)),
            pl.BlockSpec((_BR, _D), lambda i: (i, 0)),
            pl.BlockSpec((_BR, 1), lambda i: (i, 0)),
            pl.BlockSpec((_BR, _D), lambda i: (i, 0)),
            pl.BlockSpec((_D, _D), lambda i: (0, 0)),
            pl.BlockSpec((1, _D), lambda i: (0, 0)),
            pl.BlockSpec((_D, _D), lambda i: (0, 0)),
            pl.BlockSpec((_N, 1), lambda i: (0, 0)),
        ],
        out_specs=pl.BlockSpec((_BR, _D), lambda i: (i, 0)),
        out_shape=jax.ShapeDtypeStruct((_N, _D), jnp.float32),
    )(acc, cnt3, x, f, aggregated_neighbors, W_neigh, b2, W_root, f)
    return out


# speedup vs baseline: 1.0302x; 1.0302x over previous
import jax
import jax.numpy as jnp
from jax import lax
from jax.experimental import pallas as pl
from jax.experimental.pallas import tpu as pltpu
from jax.experimental.pallas import tpu_sc as plsc

_N = 10000
_E = 320000
_D = 128

_NC = 2
_NS = 16
_NWK = _NC * _NS
_CH = 128
_NCHK = _E // _CH
_CREM = _NCHK % _NWK
_CBASE = _NCHK // _NWK
_CMAX = _CBASE + 1
_BLK = 4
_PCAP = 80 * _CH
_NP = 10240
_RPT = _NP // _NS
_BR = 2000
_DUMP = _N + 64
_CROWS = _NP // 16


def _sc_compact_body(src2d_hbm, dst2d_hbm, fr_hbm, iota_hbm, csrc_hbm,
                     cdst_hbm, nch_hbm, cnt_out_hbm, fbuf, sall, dall, osrc,
                     odst, cbuf, iotab, cvec, cnt2):
    cid = lax.axis_index("c")
    sid = lax.axis_index("s")
    wid = sid * _NC + cid
    c0 = wid * _CBASE + lax.min(wid, _CREM)
    cnt = _CBASE + jnp.where(wid < _CREM, 1, 0)
    r0c = sid * (_CROWS // _NS)

    pltpu.sync_copy(fr_hbm, fbuf)
    pltpu.sync_copy(iota_hbm, iotab)
    pltpu.sync_copy(src2d_hbm.at[pl.ds(c0, _CMAX)], sall)
    pltpu.sync_copy(dst2d_hbm.at[pl.ds(c0, _CMAX)], dall)

    zsv = jnp.zeros((16,), jnp.float32)

    def zero_cbuf(i, carry):
        cbuf[i] = zsv
        return carry

    lax.fori_loop(0, _CROWS, zero_cbuf, 0)
    pltpu.sync_copy(cbuf.at[pl.ds(0, _CROWS // _NS)],
                    cnt2.at[pl.ds(r0c, _CROWS // _NS)])

    lane = lax.iota(jnp.int32, 16)

    def prefill(i, carry):
        spread = (lane + i * 16) & 127
        osrc[pl.ds(i * 16, 16)] = spread
        odst[pl.ds(i * 16, 16)] = _N + spread
        return carry

    lax.fori_loop(0, _PCAP // 16, prefill, 0)
    plsc.subcore_barrier()

    onev = jnp.ones((16,), jnp.float32)

    def chunk(j, cur):
        for t in range(_CH // 16):
            sv = sall[j, pl.ds(t * 16, 16)]
            dv = dall[j, pl.ds(t * 16, 16)]
            plsc.addupdate_scatter(cbuf, [dv >> 4, dv & 15], onev)
            fv = plsc.load_gather(fbuf, [sv])
            m = fv == 0
            plsc.store_compressed(osrc.at[pl.ds(cur, 16)], sv, mask=m)
            plsc.store_compressed(odst.at[pl.ds(cur, 16)], dv, mask=m)
            cur = cur + jnp.max(plsc.all_reduce_population_count(m))
        return cur

    cur = lax.fori_loop(0, cnt, chunk, jnp.int32(0))
    nblk = (cur + _BLK * _CH - 1) // (_BLK * _CH)

    for k in range(_CROWS // _CH):
        pltpu.sync_copy(cbuf.at[pl.ds(k * _CH, _CH)], cnt2.at[iotab.at[k]],
                        add=True)

    pltpu.sync_copy(osrc, csrc_hbm.at[wid])
    pltpu.sync_copy(odst, cdst_hbm.at[wid])
    cvec[...] = jnp.broadcast_to(nblk, (16,))
    pltpu.sync_copy(cvec, nch_hbm.at[wid])

    plsc.subcore_barrier()
    pltpu.sync_copy(cnt2.at[pl.ds(r0c, _CROWS // _NS)],
                    cnt_out_hbm.at[cid, pl.ds(r0c, _CROWS // _NS)])


def _sc_scatter_body(x_hbm, csrc_hbm, cdst_hbm, nch_hbm, zeros_hbm,
                     acc_out_hbm, sblks, dblks, rows, cvec,
                     isems, gsems, ssems, acc):
    cid = lax.axis_index("c")
    sid = lax.axis_index("s")
    wid = sid * _NC + cid
    r0 = sid * _RPT
    pltpu.sync_copy(zeros_hbm.at[pl.ds(r0, _RPT)], acc.at[pl.ds(r0, _RPT)])
    pltpu.sync_copy(nch_hbm.at[wid], cvec)
    plsc.subcore_barrier()

    nblk = jnp.max(cvec[...])

    def fetch(b, p):
        o = pl.multiple_of(b * _BLK * _CH, 8)
        pltpu.async_copy(csrc_hbm.at[wid, pl.ds(o, _BLK * _CH)], sblks[p],
                         isems[2 * p])
        pltpu.async_copy(cdst_hbm.at[wid, pl.ds(o, _BLK * _CH)], dblks[p],
                         isems[2 * p + 1])

    def fwait(p):
        pltpu.make_async_copy(csrc_hbm.at[wid, pl.ds(0, _BLK * _CH)],
                              sblks[p], isems[2 * p]).wait()
        pltpu.make_async_copy(cdst_hbm.at[wid, pl.ds(0, _BLK * _CH)],
                              dblks[p], isems[2 * p + 1]).wait()

    def process(p):
        sblk, dblk = sblks[p], dblks[p]

        def gather(u, q):
            return pltpu.async_copy(x_hbm.at[sblk.at[pl.ds(u * _CH, _CH)]],
                                    rows[q], gsems[q])

        def scatter(u, q):
            return pltpu.async_copy(rows[q], acc.at[dblk.at[pl.ds(u * _CH, _CH)]],
                                    ssems[q], add=True)

        gd, sc = {}, {}
        gd[0] = gather(0, 0)
        gd[1] = gather(1, 1)
        for u in range(0, _BLK, 2):
            gd[u].wait()
            sc[u] = scatter(u, 0)
            gd[u + 1].wait()
            sc[u + 1] = scatter(u + 1, 1)
            if u + 3 < _BLK:
                sc[u].wait()
                gd[u + 2] = gather(u + 2, 0)
                sc[u + 1].wait()
                gd[u + 3] = gather(u + 3, 1)
        sc[_BLK - 2].wait()
        sc[_BLK - 1].wait()

    @pl.when(nblk > 0)
    def _():
        fetch(0, 0)

        def pair(i, carry):
            b0 = 2 * i
            b1 = b0 + 1
            fwait(0)
            fetch(lax.min(b1, nblk - 1), 1)
            process(0)
            fwait(1)
            fetch(lax.min(b0 + 2, nblk - 1), 0)

            @pl.when(b1 < nblk)
            def _():
                process(1)

            return carry

        lax.fori_loop(0, (nblk + 1) // 2, pair, 0)
        fwait(0)

    plsc.subcore_barrier()
    pltpu.sync_copy(acc.at[pl.ds(r0, _RPT)], acc_out_hbm.at[cid, pl.ds(r0, _RPT)])


def _combine_body(acc_ref, cnt_ref, x_ref, f_ref, agg_ref, wn_ref, b_ref,
                  wr_ref, ffull_ref, out_ref):
    summed = acc_ref[0] + acc_ref[1]
    count = cnt_ref[0] + cnt_ref[1]
    mean = summed / jnp.maximum(count, 1.0)
    f = f_ref[...]
    xz = x_ref[...] * (1.0 - f)
    agg = agg_ref[...]
    use_hybrid = jnp.max(ffull_ref[...]) > 0.0
    target = (jnp.sum(jnp.abs(agg), axis=1, keepdims=True) > 0.0) & use_hybrid
    neigh_in = jnp.where(target, agg, mean)
    root_in = jnp.where(target, 0.0, xz)
    out_ref[...] = (
        jnp.dot(neigh_in, wn_ref[...], preferred_element_type=jnp.float32)
        + b_ref[...]
        + jnp.dot(root_in, wr_ref[...], preferred_element_type=jnp.float32))


def kernel(x, edge_index, frontier_mask, aggregated_neighbors,
           W_neigh, b_neigh, W_root):
    f = frontier_mask.astype(jnp.float32).reshape(_N, 1)
    fr_i = frontier_mask.astype(jnp.int32)
    src2d = jnp.pad(edge_index[0].reshape(_NCHK, _CH), ((0, 1), (0, 0)))
    dst2d = jnp.pad(edge_index[1].reshape(_NCHK, _CH), ((0, 1), (0, 0)))
    zeros = jnp.zeros((_NP, _D), jnp.float32)
    b2 = b_neigh.reshape(1, _D)

    mesh = plsc.VectorSubcoreMesh(core_axis_name="c", subcore_axis_name="s")
    params = pltpu.CompilerParams(use_tc_tiling_on_sc=False,
                                  needs_layout_passes=False)
    sc_compact = pl.kernel(
        _sc_compact_body,
        mesh=mesh,
        compiler_params=params,
        out_type=[jax.ShapeDtypeStruct((_NWK, _PCAP), jnp.int32),
                  jax.ShapeDtypeStruct((_NWK, _PCAP), jnp.int32),
                  jax.ShapeDtypeStruct((_NWK, 16), jnp.int32),
                  jax.ShapeDtypeStruct((_NC, _CROWS, 16), jnp.float32)],
        scratch_types=[
            pltpu.VMEM((_N,), jnp.int32),
            pltpu.VMEM((_CMAX, _CH), jnp.int32),
            pltpu.VMEM((_CMAX, _CH), jnp.int32),
            pltpu.VMEM((_PCAP,), jnp.int32),
            pltpu.VMEM((_PCAP,), jnp.int32),
            pltpu.VMEM((_CROWS, 16), jnp.float32),
            pltpu.VMEM((_CROWS // _CH, _CH), jnp.int32),
            pltpu.VMEM((16,), jnp.int32),
            pltpu.VMEM_SHARED((_CROWS, 16), jnp.float32),
        ],
    )
    iota_rows = jnp.arange(_CROWS, dtype=jnp.int32).reshape(_CROWS // _CH, _CH)
    csrc, cdst, nch, cnt2 = sc_compact(src2d, dst2d, fr_i, iota_rows)

    sc_scatter = pl.kernel(
        _sc_scatter_body,
        mesh=mesh,
        compiler_params=params,
        out_type=jax.ShapeDtypeStruct((_NC, _NP, _D), jnp.float32),
        scratch_types=[
            [pltpu.VMEM((_BLK * _CH,), jnp.int32) for _ in range(2)],
            [pltpu.VMEM((_BLK * _CH,), jnp.int32) for _ in range(2)],
            [pltpu.VMEM((_CH, _D), jnp.float32) for _ in range(2)],
            pltpu.VMEM((16,), jnp.int32),
            [pltpu.SemaphoreType.DMA for _ in range(4)],
            [pltpu.SemaphoreType.DMA for _ in range(2)],
            [pltpu.SemaphoreType.DMA for _ in range(2)],
            pltpu.VMEM_SHARED((_NP, _D), jnp.float32),
        ],
    )
    acc = sc_scatter(x, csrc, cdst, nch, zeros)

    cnt3 = cnt2.reshape(_NC, _NP, 1)
    out = pl.pallas_call(
        _combine_body,
        grid=(_N // _BR,),
        in_specs=[
            pl.BlockSpec((_NC, _BR, _D), lambda i: (0, i, 0)),
            pl.BlockSpec((_NC, _BR, 1), lambda i: (0, i, 0)),
            pl.BlockSpec((_BR, _D), lambda i: (i, 0)),
            pl.BlockSpec((_BR, 1), lambda i: (i, 0)),
            pl.BlockSpec((_BR, _D), lambda i: (i, 0)),
            pl.BlockSpec((_D, _D), lambda i: (0, 0)),
            pl.BlockSpec((1, _D), lambda i: (0, 0)),
            pl.BlockSpec((_D, _D), lambda i: (0, 0)),
            pl.BlockSpec((_N, 1), lambda i: (0, 0)),
        ],
        out_specs=pl.BlockSpec((_BR, _D), lambda i: (i, 0)),
        out_shape=jax.ShapeDtypeStruct((_N, _D), jnp.float32),
    )(acc, cnt3, x, f, aggregated_neighbors, W_neigh, b2, W_root, f)
    return out
